# baseline (speedup 1.0000x reference)
import math

import jax
import jax.numpy as jnp
import numpy as np
from jax.experimental import pallas as pl
from jax.experimental.pallas import tpu as pltpu

_B, _T, _H, _W, _C = 8, 16, 224, 224, 8
_F = _B * _T
_H1, _C1 = 56, 32
_H2, _C2 = 14, 128
_LIN_IN = _H2 * _H2 * _C2
_D = 512
_NF = _T // 2
_NSEG = 6
_EPS = 1e-5


def _gelu(v):
    return 0.5 * v * (1.0 + jax.lax.erf(v * (1.0 / math.sqrt(2.0))))


def _ln(v, g, b):
    mu = jnp.mean(v, axis=-1, keepdims=True)
    var = jnp.mean((v - mu) ** 2, axis=-1, keepdims=True)
    return (v - mu) * jax.lax.rsqrt(var + _EPS) * g + b


def _stage_a(x_ref, w1_ref, b1t_ref, g1t_ref, gb1t_ref, m32_ref, w2_ref,
             b2_ref, g2_ref, gb2_ref, out_ref):
    xq = x_ref[0]
    m32 = m32_ref[...]
    outs = []
    for j in range(_H2):
        o3 = jnp.zeros((_H2, _C2), jnp.float32) + b2_ref[...]
        for qh in range(4):
            base = qh * 7168 + j * 128
            c = sum(
                jnp.dot(xq[:, base + ph * 1792: base + ph * 1792 + 128],
                        w1_ref[ph], preferred_element_type=jnp.float32)
                for ph in range(4))
            c = c + b1t_ref[...]
            mu = jnp.dot(c, m32, preferred_element_type=jnp.float32)
            ex2 = jnp.dot(c * c, m32, preferred_element_type=jnp.float32)
            var = ex2 - mu * mu
            y = ((c - mu) * jax.lax.rsqrt(var + _EPS) * g1t_ref[...]
                 + gb1t_ref[...])
            y = _gelu(y)
            o3 = o3 + jnp.dot(y, w2_ref[qh],
                              preferred_element_type=jnp.float32)
        outs.append(_gelu(_ln(o3, g2_ref[...], gb2_ref[...])))
    out_ref[0] = jnp.concatenate(outs, axis=-1)


def _stage_b(a_ref, w_ref, b_ref, out_ref):
    k = pl.program_id(0)

    @pl.when(k == 0)
    def _():
        out_ref[...] = jnp.broadcast_to(b_ref[...], out_ref.shape)

    out_ref[...] += jnp.dot(a_ref[...], w_ref[...],
                            preferred_element_type=jnp.float32)


def _stage_c(h_ref, cre_ref, cim_ref, wg_ref, out_ref):
    lane = jax.lax.broadcasted_iota(jnp.int32, (1, 8), 1)
    for b in range(_B):
        hb = h_ref[b * _T:(b + 1) * _T, :]
        re = jnp.dot(cre_ref[...], hb, preferred_element_type=jnp.float32)
        im = jnp.dot(cim_ref[...], hb, preferred_element_type=jnp.float32)
        amp = jnp.sqrt(re * re + im * im)
        ampmean = jnp.mean(amp, axis=1, keepdims=True)
        logits = jnp.sum(ampmean * wg_ref[...], axis=0, keepdims=True)
        logits = jnp.where(lane < _NSEG, logits, -1e30)
        m1 = jnp.max(logits)
        i1 = jnp.argmax(logits, axis=1)[0]
        masked = jnp.where(lane == i1, -1e30, logits)
        m2 = jnp.max(masked)
        i2 = jnp.argmax(masked, axis=1)[0]
        e = jnp.exp(m2 - m1)
        gtop = 1.0 / (1.0 + e)
        gsec = e / (1.0 + e)
        row = jnp.where(lane == i1, gtop,
                        jnp.where(lane == i2, gsec, 0.0))
        out_ref[pl.ds(b, 1), :] = row


@jax.jit
def kernel(x, conv1_w, conv1_b, ln1_g, ln1_b, conv2_w, conv2_b, ln2_g,
           ln2_b, lin_w, lin_b, w_gate):
    xv = x.reshape(_F, _H2, 16 * _W * _C)
    eye4 = jnp.eye(4, dtype=jnp.float32)
    w1s = conv1_w.reshape(4, _C1, _C1)
    w1bd = (eye4[None, :, None, :, None] * w1s[:, None, :, None, :]
            ).reshape(4, 4 * _C1, 4 * _C1)
    w2 = conv2_w.reshape(4, 4 * _C1, _C2)
    tile4 = lambda v: jnp.tile(v, 4).reshape(1, 4 * _C1)
    m32 = jnp.asarray(np.kron(np.eye(4), np.ones((_C1, _C1)) / _C1),
                      jnp.float32)

    feat = pl.pallas_call(
        _stage_a,
        grid=(_F,),
        in_specs=[
            pl.BlockSpec((1, _H2, 16 * _W * _C), lambda i: (i, 0, 0)),
            pl.BlockSpec(w1bd.shape, lambda i: (0, 0, 0)),
            pl.BlockSpec((1, _C2), lambda i: (0, 0)),
            pl.BlockSpec((1, _C2), lambda i: (0, 0)),
            pl.BlockSpec((1, _C2), lambda i: (0, 0)),
            pl.BlockSpec(m32.shape, lambda i: (0, 0)),
            pl.BlockSpec(w2.shape, lambda i: (0, 0, 0)),
            pl.BlockSpec((1, _C2), lambda i: (0, 0)),
            pl.BlockSpec((1, _C2), lambda i: (0, 0)),
            pl.BlockSpec((1, _C2), lambda i: (0, 0)),
        ],
        out_specs=pl.BlockSpec((1, _H2, _H2 * _C2), lambda i: (i, 0, 0)),
        out_shape=jax.ShapeDtypeStruct((_F, _H2, _H2 * _C2), jnp.float32),
    )(xv, w1bd, tile4(conv1_b), tile4(ln1_g), tile4(ln1_b), m32,
      w2, conv2_b.reshape(1, _C2), ln2_g.reshape(1, _C2),
      ln2_b.reshape(1, _C2))

    featf = feat.reshape(_F, _LIN_IN)
    kb = 3584
    nk = _LIN_IN // kb
    h = pl.pallas_call(
        _stage_b,
        grid=(nk,),
        in_specs=[
            pl.BlockSpec((_F, kb), lambda k: (0, k)),
            pl.BlockSpec((kb, _D), lambda k: (k, 0)),
            pl.BlockSpec((1, _D), lambda k: (0, 0)),
        ],
        out_specs=pl.BlockSpec((_F, _D), lambda k: (0, 0)),
        out_shape=jax.ShapeDtypeStruct((_F, _D), jnp.float32),
    )(featf, lin_w, lin_b.reshape(1, _D))

    t = np.arange(_T)[:, None]
    f = np.arange(1, _NF + 1)[None, :]
    ang = 2.0 * np.pi * t * f / _T
    scale = 1.0 / np.sqrt(_T)
    cre = jnp.asarray((np.cos(ang) * scale).T, jnp.float32)
    cim = jnp.asarray((-np.sin(ang) * scale).T, jnp.float32)
    wg = jnp.pad(w_gate, ((0, 0), (0, 8 - _NSEG)))

    gates = pl.pallas_call(
        _stage_c,
        grid=(1,),
        in_specs=[
            pl.BlockSpec((_F, _D), lambda i: (0, 0)),
            pl.BlockSpec((_NF, _T), lambda i: (0, 0)),
            pl.BlockSpec((_NF, _T), lambda i: (0, 0)),
            pl.BlockSpec((_NF, 8), lambda i: (0, 0)),
        ],
        out_specs=pl.BlockSpec((_B, 8), lambda i: (0, 0)),
        out_shape=jax.ShapeDtypeStruct((_B, 8), jnp.float32),
    )(h, cre, cim, wg)

    return gates[:, :_NSEG]


# speedup vs baseline: 1.8797x; 1.8797x over previous
import math

import jax
import jax.numpy as jnp
import numpy as np
from jax.experimental import pallas as pl
from jax.experimental.pallas import tpu as pltpu

_B, _T, _H, _W, _C = 8, 16, 224, 224, 8
_F = _B * _T
_H1, _C1 = 56, 32
_H2, _C2 = 14, 128
_LIN_IN = _H2 * _H2 * _C2
_D = 512
_NF = _T // 2
_NSEG = 6
_EPS = 1e-5


def _gelu(v):
    return 0.5 * v * (1.0 + jax.lax.erf(v * (1.0 / math.sqrt(2.0))))


def _ln(v, g, b):
    mu = jnp.mean(v, axis=-1, keepdims=True)
    var = jnp.mean((v - mu) ** 2, axis=-1, keepdims=True)
    return (v - mu) * jax.lax.rsqrt(var + _EPS) * g + b


def _stage_a(x_ref, w1_ref, b1t_ref, g1t_ref, gb1t_ref, m32_ref, w2_ref,
             b2_ref, g2_ref, gb2_ref, m128_ref, out_ref):
    xq = x_ref[...]
    m32 = m32_ref[...]
    m128 = m128_ref[...]
    for j in range(_H2):
        ys = []
        for qh in range(4):
            base = qh * 7168 + j * 128
            xcat = jnp.concatenate(
                [xq[:, base + ph * 1792: base + ph * 1792 + 128]
                 for ph in range(4)], axis=1)
            c = jnp.dot(xcat, w1_ref[...],
                        preferred_element_type=jnp.float32) + b1t_ref[...]
            mu = jnp.dot(c, m32, preferred_element_type=jnp.float32)
            ex2 = jnp.dot(c * c, m32, preferred_element_type=jnp.float32)
            var = ex2 - mu * mu
            y = ((c - mu) * jax.lax.rsqrt(var + _EPS) * g1t_ref[...]
                 + gb1t_ref[...])
            ys.append(_gelu(y))
        ycat = jnp.concatenate(ys, axis=1)
        o = jnp.dot(ycat, w2_ref[...],
                    preferred_element_type=jnp.float32) + b2_ref[...]
        mu2 = jnp.dot(o, m128, preferred_element_type=jnp.float32)
        ex22 = jnp.dot(o * o, m128, preferred_element_type=jnp.float32)
        y2 = ((o - mu2) * jax.lax.rsqrt(ex22 - mu2 * mu2 + _EPS)
              * g2_ref[...] + gb2_ref[...])
        out_ref[:, j * _C2:(j + 1) * _C2] = _gelu(y2)


def _stage_b(a_ref, w_ref, b_ref, out_ref):
    k = pl.program_id(0)

    @pl.when(k == 0)
    def _():
        out_ref[...] = jnp.broadcast_to(b_ref[...], out_ref.shape)

    out_ref[...] += jnp.dot(a_ref[...], w_ref[...],
                            preferred_element_type=jnp.float32)


def _stage_c(h_ref, cre_ref, cim_ref, wg_ref, out_ref):
    lane = jax.lax.broadcasted_iota(jnp.int32, (1, 8), 1)
    for b in range(_B):
        hb = h_ref[b * _T:(b + 1) * _T, :]
        re = jnp.dot(cre_ref[...], hb, preferred_element_type=jnp.float32)
        im = jnp.dot(cim_ref[...], hb, preferred_element_type=jnp.float32)
        amp = jnp.sqrt(re * re + im * im)
        ampmean = jnp.mean(amp, axis=1, keepdims=True)
        logits = jnp.sum(ampmean * wg_ref[...], axis=0, keepdims=True)
        logits = jnp.where(lane < _NSEG, logits, -1e30)
        m1 = jnp.max(logits)
        i1 = jnp.argmax(logits, axis=1)[0]
        masked = jnp.where(lane == i1, -1e30, logits)
        m2 = jnp.max(masked)
        i2 = jnp.argmax(masked, axis=1)[0]
        e = jnp.exp(m2 - m1)
        gtop = 1.0 / (1.0 + e)
        gsec = e / (1.0 + e)
        row = jnp.where(lane == i1, gtop,
                        jnp.where(lane == i2, gsec, 0.0))
        out_ref[pl.ds(b, 1), :] = row


@jax.jit
def kernel(x, conv1_w, conv1_b, ln1_g, ln1_b, conv2_w, conv2_b, ln2_g,
           ln2_b, lin_w, lin_b, w_gate):
    xv = x.reshape(_F, _H2 * 16 * _W * _C)
    eye4 = jnp.eye(4, dtype=jnp.float32)
    w1s = conv1_w.reshape(4, _C1, _C1)
    w1bd = (eye4[None, :, None, :, None] * w1s[:, None, :, None, :]
            ).reshape(4, 4 * _C1, 4 * _C1)
    w2 = conv2_w.reshape(4, 4 * _C1, _C2)
    w1cat = w1bd.reshape(4 * 4 * _C1, 4 * _C1)
    w2cat = w2.reshape(4 * 4 * _C1, _C2)
    tile4 = lambda v: jnp.tile(v, 4).reshape(1, 4 * _C1)
    m32 = jnp.asarray(np.kron(np.eye(4), np.ones((_C1, _C1)) / _C1),
                      jnp.float32)
    m128 = jnp.full((_C2, _C2), 1.0 / _C2, jnp.float32)

    feat = pl.pallas_call(
        _stage_a,
        grid=(_H2,),
        in_specs=[
            pl.BlockSpec((_F, 16 * _W * _C), lambda i: (0, i)),
            pl.BlockSpec(w1cat.shape, lambda i: (0, 0)),
            pl.BlockSpec((1, _C2), lambda i: (0, 0)),
            pl.BlockSpec((1, _C2), lambda i: (0, 0)),
            pl.BlockSpec((1, _C2), lambda i: (0, 0)),
            pl.BlockSpec(m32.shape, lambda i: (0, 0)),
            pl.BlockSpec(w2cat.shape, lambda i: (0, 0)),
            pl.BlockSpec((1, _C2), lambda i: (0, 0)),
            pl.BlockSpec((1, _C2), lambda i: (0, 0)),
            pl.BlockSpec((1, _C2), lambda i: (0, 0)),
            pl.BlockSpec(m128.shape, lambda i: (0, 0)),
        ],
        out_specs=pl.BlockSpec((_F, _H2 * _C2), lambda i: (0, i)),
        out_shape=jax.ShapeDtypeStruct((_F, _LIN_IN), jnp.float32),
    )(xv, w1cat, tile4(conv1_b), tile4(ln1_g), tile4(ln1_b), m32,
      w2cat, conv2_b.reshape(1, _C2), ln2_g.reshape(1, _C2),
      ln2_b.reshape(1, _C2), m128)

    featf = feat
    kb = 3584
    nk = _LIN_IN // kb
    h = pl.pallas_call(
        _stage_b,
        grid=(nk,),
        in_specs=[
            pl.BlockSpec((_F, kb), lambda k: (0, k)),
            pl.BlockSpec((kb, _D), lambda k: (k, 0)),
            pl.BlockSpec((1, _D), lambda k: (0, 0)),
        ],
        out_specs=pl.BlockSpec((_F, _D), lambda k: (0, 0)),
        out_shape=jax.ShapeDtypeStruct((_F, _D), jnp.float32),
    )(featf, lin_w, lin_b.reshape(1, _D))

    t = np.arange(_T)[:, None]
    f = np.arange(1, _NF + 1)[None, :]
    ang = 2.0 * np.pi * t * f / _T
    scale = 1.0 / np.sqrt(_T)
    cre = jnp.asarray((np.cos(ang) * scale).T, jnp.float32)
    cim = jnp.asarray((-np.sin(ang) * scale).T, jnp.float32)
    wg = jnp.pad(w_gate, ((0, 0), (0, 8 - _NSEG)))

    gates = pl.pallas_call(
        _stage_c,
        grid=(1,),
        in_specs=[
            pl.BlockSpec((_F, _D), lambda i: (0, 0)),
            pl.BlockSpec((_NF, _T), lambda i: (0, 0)),
            pl.BlockSpec((_NF, _T), lambda i: (0, 0)),
            pl.BlockSpec((_NF, 8), lambda i: (0, 0)),
        ],
        out_specs=pl.BlockSpec((_B, 8), lambda i: (0, 0)),
        out_shape=jax.ShapeDtypeStruct((_B, 8), jnp.float32),
    )(h, cre, cim, wg)

    return gates[:, :_NSEG]


# speedup vs baseline: 2.1004x; 1.1174x over previous
import math

import jax
import jax.numpy as jnp
import numpy as np
from jax.experimental import pallas as pl
from jax.experimental.pallas import tpu as pltpu

_B, _T, _H, _W, _C = 8, 16, 224, 224, 8
_F = _B * _T
_H1, _C1 = 56, 32
_H2, _C2 = 14, 128
_LIN_IN = _H2 * _H2 * _C2
_D = 512
_NF = _T // 2
_NSEG = 6
_EPS = 1e-5


def _gelu(v):
    return 0.5 * v * (1.0 + jax.lax.erf(v * (1.0 / math.sqrt(2.0))))


def _ln(v, g, b):
    mu = jnp.mean(v, axis=-1, keepdims=True)
    var = jnp.mean((v - mu) ** 2, axis=-1, keepdims=True)
    return (v - mu) * jax.lax.rsqrt(var + _EPS) * g + b


def _stage_a(x_ref, w1_ref, b1t_ref, g1t_ref, gb1t_ref, m32_ref, w2_ref,
             b2_ref, g2_ref, gb2_ref, m128_ref, out_ref):
    xq = x_ref[...]
    m32 = m32_ref[...]
    m128 = m128_ref[...]
    for j in range(_H2):
        ys = []
        for qh in range(4):
            base = qh * 7168 + j * 128
            xcat = jnp.concatenate(
                [xq[base + ph * 1792: base + ph * 1792 + 128, :]
                 for ph in range(4)], axis=0)
            c = jax.lax.dot_general(
                xcat, w1_ref[...],
                dimension_numbers=(((0,), (0,)), ((), ())),
                preferred_element_type=jnp.float32) + b1t_ref[...]
            mu = jnp.dot(c, m32, preferred_element_type=jnp.float32)
            ex2 = jnp.dot(c * c, m32, preferred_element_type=jnp.float32)
            var = ex2 - mu * mu
            y = ((c - mu) * jax.lax.rsqrt(var + _EPS) * g1t_ref[...]
                 + gb1t_ref[...])
            ys.append(_gelu(y))
        ycat = jnp.concatenate(ys, axis=1)
        o = jnp.dot(ycat, w2_ref[...],
                    preferred_element_type=jnp.float32) + b2_ref[...]
        mu2 = jnp.dot(o, m128, preferred_element_type=jnp.float32)
        ex22 = jnp.dot(o * o, m128, preferred_element_type=jnp.float32)
        y2 = ((o - mu2) * jax.lax.rsqrt(ex22 - mu2 * mu2 + _EPS)
              * g2_ref[...] + gb2_ref[...])
        out_ref[:, j * _C2:(j + 1) * _C2] = _gelu(y2)


def _stage_b(a_ref, w_ref, b_ref, out_ref):
    k = pl.program_id(0)

    @pl.when(k == 0)
    def _():
        out_ref[...] = jnp.broadcast_to(b_ref[...], out_ref.shape)

    out_ref[...] += jnp.dot(a_ref[...], w_ref[...],
                            preferred_element_type=jnp.float32)


def _stage_c(h_ref, cre_ref, cim_ref, wg_ref, out_ref):
    lane = jax.lax.broadcasted_iota(jnp.int32, (1, 8), 1)
    for b in range(_B):
        hb = h_ref[b * _T:(b + 1) * _T, :]
        re = jnp.dot(cre_ref[...], hb, preferred_element_type=jnp.float32)
        im = jnp.dot(cim_ref[...], hb, preferred_element_type=jnp.float32)
        amp = jnp.sqrt(re * re + im * im)
        ampmean = jnp.mean(amp, axis=1, keepdims=True)
        logits = jnp.sum(ampmean * wg_ref[...], axis=0, keepdims=True)
        logits = jnp.where(lane < _NSEG, logits, -1e30)
        m1 = jnp.max(logits)
        i1 = jnp.argmax(logits, axis=1)[0]
        masked = jnp.where(lane == i1, -1e30, logits)
        m2 = jnp.max(masked)
        i2 = jnp.argmax(masked, axis=1)[0]
        e = jnp.exp(m2 - m1)
        gtop = 1.0 / (1.0 + e)
        gsec = e / (1.0 + e)
        row = jnp.where(lane == i1, gtop,
                        jnp.where(lane == i2, gsec, 0.0))
        out_ref[pl.ds(b, 1), :] = row


@jax.jit
def kernel(x, conv1_w, conv1_b, ln1_g, ln1_b, conv2_w, conv2_b, ln2_g,
           ln2_b, lin_w, lin_b, w_gate):
    xv = jnp.transpose(x, (2, 3, 4, 0, 1)).reshape(_H2 * 16 * _W * _C, _F)
    eye4 = jnp.eye(4, dtype=jnp.float32)
    w1s = conv1_w.reshape(4, _C1, _C1)
    w1bd = (eye4[None, :, None, :, None] * w1s[:, None, :, None, :]
            ).reshape(4, 4 * _C1, 4 * _C1)
    w2 = conv2_w.reshape(4, 4 * _C1, _C2)
    w1cat = w1bd.reshape(4 * 4 * _C1, 4 * _C1)
    w2cat = w2.reshape(4 * 4 * _C1, _C2)
    tile4 = lambda v: jnp.tile(v, 4).reshape(1, 4 * _C1)
    m32 = jnp.asarray(np.kron(np.eye(4), np.ones((_C1, _C1)) / _C1),
                      jnp.float32)
    m128 = jnp.full((_C2, _C2), 1.0 / _C2, jnp.float32)

    feat = pl.pallas_call(
        _stage_a,
        grid=(_H2,),
        in_specs=[
            pl.BlockSpec((16 * _W * _C, _F), lambda i: (i, 0)),
            pl.BlockSpec(w1cat.shape, lambda i: (0, 0)),
            pl.BlockSpec((1, _C2), lambda i: (0, 0)),
            pl.BlockSpec((1, _C2), lambda i: (0, 0)),
            pl.BlockSpec((1, _C2), lambda i: (0, 0)),
            pl.BlockSpec(m32.shape, lambda i: (0, 0)),
            pl.BlockSpec(w2cat.shape, lambda i: (0, 0)),
            pl.BlockSpec((1, _C2), lambda i: (0, 0)),
            pl.BlockSpec((1, _C2), lambda i: (0, 0)),
            pl.BlockSpec((1, _C2), lambda i: (0, 0)),
            pl.BlockSpec(m128.shape, lambda i: (0, 0)),
        ],
        out_specs=pl.BlockSpec((_F, _H2 * _C2), lambda i: (0, i)),
        out_shape=jax.ShapeDtypeStruct((_F, _LIN_IN), jnp.float32),
    )(xv, w1cat, tile4(conv1_b), tile4(ln1_g), tile4(ln1_b), m32,
      w2cat, conv2_b.reshape(1, _C2), ln2_g.reshape(1, _C2),
      ln2_b.reshape(1, _C2), m128)

    featf = feat
    kb = 3584
    nk = _LIN_IN // kb
    h = pl.pallas_call(
        _stage_b,
        grid=(nk,),
        in_specs=[
            pl.BlockSpec((_F, kb), lambda k: (0, k)),
            pl.BlockSpec((kb, _D), lambda k: (k, 0)),
            pl.BlockSpec((1, _D), lambda k: (0, 0)),
        ],
        out_specs=pl.BlockSpec((_F, _D), lambda k: (0, 0)),
        out_shape=jax.ShapeDtypeStruct((_F, _D), jnp.float32),
    )(featf, lin_w, lin_b.reshape(1, _D))

    t = np.arange(_T)[:, None]
    f = np.arange(1, _NF + 1)[None, :]
    ang = 2.0 * np.pi * t * f / _T
    scale = 1.0 / np.sqrt(_T)
    cre = jnp.asarray((np.cos(ang) * scale).T, jnp.float32)
    cim = jnp.asarray((-np.sin(ang) * scale).T, jnp.float32)
    wg = jnp.pad(w_gate, ((0, 0), (0, 8 - _NSEG)))

    gates = pl.pallas_call(
        _stage_c,
        grid=(1,),
        in_specs=[
            pl.BlockSpec((_F, _D), lambda i: (0, 0)),
            pl.BlockSpec((_NF, _T), lambda i: (0, 0)),
            pl.BlockSpec((_NF, _T), lambda i: (0, 0)),
            pl.BlockSpec((_NF, 8), lambda i: (0, 0)),
        ],
        out_specs=pl.BlockSpec((_B, 8), lambda i: (0, 0)),
        out_shape=jax.ShapeDtypeStruct((_B, 8), jnp.float32),
    )(h, cre, cim, wg)

    return gates[:, :_NSEG]
